# baseline (speedup 1.0000x reference)
import numpy as np

import jax
import jax.numpy as jnp
from jax.experimental import pallas as pl
from jax.experimental.pallas import tpu as pltpu

_ROT_A = (13, 15, 26, 6)
_ROT_B = (17, 29, 16, 24)
_TINY = np.float32(np.finfo(np.float32).tiny)
_ONE = np.float32(1.0)
_NEG_INF = np.float32(-np.inf)
_LOWEST = np.float32(np.finfo(np.float32).min)
_BIG = np.int32(0x7FFFFFFF)


def _tf_rounds(x0, x1, rots):
    for r in rots:
        x0 = x0 + x1
        x1 = (x1 << jnp.uint32(r)) | (x1 >> jnp.uint32(32 - r))
        x1 = x0 ^ x1
    return x0, x1


def _threefry_bits(k1, k2, idx):
    ks2 = k1 ^ k2 ^ jnp.uint32(0x1BD11BDA)
    x0 = jnp.zeros_like(idx) + k1
    x1 = idx + k2
    x0, x1 = _tf_rounds(x0, x1, _ROT_A)
    x0 = x0 + k2
    x1 = x1 + ks2 + jnp.uint32(1)
    x0, x1 = _tf_rounds(x0, x1, _ROT_B)
    x0 = x0 + ks2
    x1 = x1 + k1 + jnp.uint32(2)
    x0, x1 = _tf_rounds(x0, x1, _ROT_A)
    x0 = x0 + k1
    x1 = x1 + k2 + jnp.uint32(3)
    x0, x1 = _tf_rounds(x0, x1, _ROT_B)
    x0 = x0 + k2
    x1 = x1 + ks2 + jnp.uint32(4)
    x0, x1 = _tf_rounds(x0, x1, _ROT_A)
    x0 = x0 + ks2
    x1 = x1 + k1 + jnp.uint32(5)
    return x0 ^ x1


def _chunk_stats(x, safe_t, k1, k2, base, vocab, rows, block):
    scaled = x / safe_t
    col = jax.lax.broadcasted_iota(jnp.int32, (block, rows), 0) + base
    row = jax.lax.broadcasted_iota(jnp.int32, (block, rows), 1)
    idx = (row * vocab + col).astype(jnp.uint32)
    bits = _threefry_bits(k1, k2, idx)
    fbits = (bits >> jnp.uint32(9)) | jnp.uint32(0x3F800000)
    u01 = jax.lax.bitcast_convert_type(fbits, jnp.float32) - _ONE
    u = jnp.maximum(_TINY, u01 * (_ONE - _TINY) + _TINY)
    g = -jnp.log(-jnp.log(u))

    sm = jnp.where(col < vocab, scaled, _NEG_INF)
    ym = sm + g

    bm = jnp.maximum(jnp.max(sm, axis=0, keepdims=True), _LOWEST)
    bxi = jnp.min(jnp.where(sm == bm, col, _BIG), axis=0, keepdims=True)
    by = jnp.max(ym, axis=0, keepdims=True)
    byi = jnp.min(jnp.where(ym == by, col, _BIG), axis=0, keepdims=True)
    bysel = jnp.max(jnp.where(col == byi, sm, _NEG_INF), axis=0, keepdims=True)
    bs = jnp.sum(jnp.exp(sm - bm), axis=0, keepdims=True)
    return (bm, bxi, by, byi, bysel, bs)


def _merge(a, b):
    am, axi, ay, ayi, aysel, as_ = a
    bm, bxi, by, byi, bysel, bs = b
    upg = bm > am
    m_new = jnp.where(upg, bm, am)
    gxi = jnp.where(upg, bxi, axi)
    s_new = as_ * jnp.exp(am - m_new) + bs * jnp.exp(bm - m_new)
    upy = by > ay
    return (m_new, gxi,
            jnp.where(upy, by, ay), jnp.where(upy, byi, ayi),
            jnp.where(upy, bysel, aysel), s_new)


def _sampler_body(vocab, block, nchunk, nsteps, kd_ref, t_ref, lt_ref,
                  tok_ref, lp_ref, gidx, ymax, yidx, ysel, lm, ls):
    j = pl.program_id(0)
    rows = lt_ref.shape[1]
    t = t_ref[...]
    safe_t = jnp.where(t == 0.0, _ONE, t)
    k1 = kd_ref[0]
    k2 = kd_ref[1]

    acc = None
    for c in range(nchunk):
        x = lt_ref[c * block:(c + 1) * block, :]
        base = j * (nchunk * block) + c * block
        st = _chunk_stats(x, safe_t, k1, k2, base, vocab, rows, block)
        acc = st if acc is None else _merge(acc, st)

    @pl.when(j == 0)
    def _init():
        lm[...], gidx[...], ymax[...], yidx[...], ysel[...], ls[...] = acc

    @pl.when(j > 0)
    def _mrg():
        carry = (lm[...], gidx[...], ymax[...], yidx[...], ysel[...], ls[...])
        m = _merge(carry, acc)
        lm[...], gidx[...], ymax[...], yidx[...], ysel[...], ls[...] = m

    @pl.when(j == nsteps - 1)
    def _finish():
        zero_t = t == 0.0
        tok_ref[...] = jnp.where(zero_t, gidx[...], yidx[...])
        sel = jnp.where(zero_t, lm[...], ysel[...])
        log_z = lm[...] + jnp.log(ls[...])
        lp_ref[...] = sel - log_z


def kernel(logits, temperatures, key):
    rows, vocab = logits.shape
    if logits.dtype != jnp.float32:
        logits = logits.astype(jnp.float32)
    if temperatures.dtype != jnp.float32:
        temperatures = temperatures.astype(jnp.float32)
    kd = jax.random.key_data(key).astype(jnp.uint32).reshape(2)
    t2 = temperatures.reshape(1, rows)
    lt = logits.T

    if vocab > 2048:
        block, nchunk = 2048, 1
    else:
        block, nchunk = max(8, -(-vocab // 8) * 8), 1
    step_rows = block * nchunk
    nsteps = -(-vocab // step_rows)

    fn = lambda *a: _sampler_body(vocab, block, nchunk, nsteps, *a)
    tok, lp = pl.pallas_call(
        fn,
        grid=(nsteps,),
        in_specs=[
            pl.BlockSpec(memory_space=pltpu.SMEM),
            pl.BlockSpec((1, rows), lambda j: (0, 0)),
            pl.BlockSpec((step_rows, rows), lambda j: (j, 0)),
        ],
        out_specs=[
            pl.BlockSpec((1, rows), lambda j: (0, 0)),
            pl.BlockSpec((1, rows), lambda j: (0, 0)),
        ],
        out_shape=[
            jax.ShapeDtypeStruct((1, rows), jnp.int32),
            jax.ShapeDtypeStruct((1, rows), jnp.float32),
        ],
        scratch_shapes=[
            pltpu.VMEM((1, rows), jnp.int32),
            pltpu.VMEM((1, rows), jnp.float32),
            pltpu.VMEM((1, rows), jnp.int32),
            pltpu.VMEM((1, rows), jnp.float32),
            pltpu.VMEM((1, rows), jnp.float32),
            pltpu.VMEM((1, rows), jnp.float32),
        ],
        compiler_params=pltpu.CompilerParams(
            dimension_semantics=("arbitrary",),
        ),
    )(kd, t2, lt)
    return tok.reshape(rows), lp.reshape(rows)


# speedup vs baseline: 1.6045x; 1.6045x over previous
import numpy as np

import jax
import jax.numpy as jnp
from jax.experimental import pallas as pl
from jax.experimental.pallas import tpu as pltpu

_ROT_A = (13, 15, 26, 6)
_ROT_B = (17, 29, 16, 24)
_TINY = np.float32(np.finfo(np.float32).tiny)
_ONE = np.float32(1.0)
_NEG_INF = np.float32(-np.inf)
_LOWEST = np.float32(np.finfo(np.float32).min)
_BIG = np.int32(0x7FFFFFFF)


def _tf_rounds(x0, x1, rots):
    for r in rots:
        x0 = x0 + x1
        x1 = (x1 << jnp.uint32(r)) | (x1 >> jnp.uint32(32 - r))
        x1 = x0 ^ x1
    return x0, x1


def _threefry_bits(k1, k2, idx):
    ks2 = k1 ^ k2 ^ jnp.uint32(0x1BD11BDA)
    x0 = jnp.zeros_like(idx) + k1
    x1 = idx + k2
    x0, x1 = _tf_rounds(x0, x1, _ROT_A)
    x0 = x0 + k2
    x1 = x1 + ks2 + jnp.uint32(1)
    x0, x1 = _tf_rounds(x0, x1, _ROT_B)
    x0 = x0 + ks2
    x1 = x1 + k1 + jnp.uint32(2)
    x0, x1 = _tf_rounds(x0, x1, _ROT_A)
    x0 = x0 + k1
    x1 = x1 + k2 + jnp.uint32(3)
    x0, x1 = _tf_rounds(x0, x1, _ROT_B)
    x0 = x0 + k2
    x1 = x1 + ks2 + jnp.uint32(4)
    x0, x1 = _tf_rounds(x0, x1, _ROT_A)
    x0 = x0 + ks2
    x1 = x1 + k1 + jnp.uint32(5)
    return x0 ^ x1


def _chunk_stats(x, safe_t, k1, k2, base, vocab, rows, block):
    scaled = x / safe_t
    col = jax.lax.broadcasted_iota(jnp.int32, (rows, block), 1) + base
    row = jax.lax.broadcasted_iota(jnp.int32, (rows, block), 0)
    idx = (row * vocab + col).astype(jnp.uint32)
    bits = _threefry_bits(k1, k2, idx)
    fbits = (bits >> jnp.uint32(9)) | jnp.uint32(0x3F800000)
    u01 = jax.lax.bitcast_convert_type(fbits, jnp.float32) - _ONE
    u = jnp.maximum(_TINY, u01 * (_ONE - _TINY) + _TINY)
    g = -jnp.log(-jnp.log(u))

    sm = jnp.where(col < vocab, scaled, _NEG_INF)
    ym = sm + g

    bm = jnp.maximum(jnp.max(sm, axis=1, keepdims=True), _LOWEST)
    bxi = jnp.min(jnp.where(sm == bm, col, _BIG), axis=1, keepdims=True)
    by = jnp.max(ym, axis=1, keepdims=True)
    byi = jnp.min(jnp.where(ym == by, col, _BIG), axis=1, keepdims=True)
    bysel = jnp.max(jnp.where(col == byi, sm, _NEG_INF), axis=1, keepdims=True)
    bs = jnp.sum(jnp.exp(sm - bm), axis=1, keepdims=True)
    return (bm, bxi, by, byi, bysel, bs)


def _merge(a, b):
    am, axi, ay, ayi, aysel, as_ = a
    bm, bxi, by, byi, bysel, bs = b
    upg = bm > am
    m_new = jnp.where(upg, bm, am)
    gxi = jnp.where(upg, bxi, axi)
    s_new = as_ * jnp.exp(am - m_new) + bs * jnp.exp(bm - m_new)
    upy = by > ay
    return (m_new, gxi,
            jnp.where(upy, by, ay), jnp.where(upy, byi, ayi),
            jnp.where(upy, bysel, aysel), s_new)


def _sampler_body(vocab, block, nchunk, nsteps, kd_ref, t_ref, lt_ref,
                  tok_ref, lp_ref, gidx, ymax, yidx, ysel, lm, ls, tcol):
    j = pl.program_id(0)
    rows = lt_ref.shape[1]
    t = t_ref[...]

    @pl.when(j == 0)
    def _stage_t():
        tcol[...] = jnp.transpose(t, (1, 0))

    t_col = tcol[...]
    safe_t = jnp.where(t_col == 0.0, _ONE, t_col)
    k1 = kd_ref[0]
    k2 = kd_ref[1]

    acc = None
    for c in range(nchunk):
        x = jnp.transpose(lt_ref[c * block:(c + 1) * block, :], (1, 0))
        base = j * (nchunk * block) + c * block
        st = _chunk_stats(x, safe_t, k1, k2, base, vocab, rows, block)
        acc = st if acc is None else _merge(acc, st)

    @pl.when(j == 0)
    def _init():
        lm[...], gidx[...], ymax[...], yidx[...], ysel[...], ls[...] = acc

    @pl.when(j > 0)
    def _mrg():
        carry = (lm[...], gidx[...], ymax[...], yidx[...], ysel[...], ls[...])
        m = _merge(carry, acc)
        lm[...], gidx[...], ymax[...], yidx[...], ysel[...], ls[...] = m

    @pl.when(j == nsteps - 1)
    def _finish():
        zero_t = t_col == 0.0
        tok = jnp.where(zero_t, gidx[...], yidx[...])
        sel = jnp.where(zero_t, lm[...], ysel[...])
        log_z = lm[...] + jnp.log(ls[...])
        tok_ref[...] = jnp.transpose(tok, (1, 0))
        lp_ref[...] = jnp.transpose(sel - log_z, (1, 0))


def kernel(logits, temperatures, key):
    rows, vocab = logits.shape
    if logits.dtype != jnp.float32:
        logits = logits.astype(jnp.float32)
    if temperatures.dtype != jnp.float32:
        temperatures = temperatures.astype(jnp.float32)
    kd = jax.random.key_data(key).astype(jnp.uint32).reshape(2)
    t2 = temperatures.reshape(1, rows)
    lt = logits.T

    if vocab > 2048:
        block, nchunk = 2048, 1
    else:
        block, nchunk = max(8, -(-vocab // 8) * 8), 1
    step_rows = block * nchunk
    nsteps = -(-vocab // step_rows)

    fn = lambda *a: _sampler_body(vocab, block, nchunk, nsteps, *a)
    tok, lp = pl.pallas_call(
        fn,
        grid=(nsteps,),
        in_specs=[
            pl.BlockSpec(memory_space=pltpu.SMEM),
            pl.BlockSpec((1, rows), lambda j: (0, 0)),
            pl.BlockSpec((step_rows, rows), lambda j: (j, 0)),
        ],
        out_specs=[
            pl.BlockSpec((1, rows), lambda j: (0, 0)),
            pl.BlockSpec((1, rows), lambda j: (0, 0)),
        ],
        out_shape=[
            jax.ShapeDtypeStruct((1, rows), jnp.int32),
            jax.ShapeDtypeStruct((1, rows), jnp.float32),
        ],
        scratch_shapes=[
            pltpu.VMEM((rows, 1), jnp.int32),
            pltpu.VMEM((rows, 1), jnp.float32),
            pltpu.VMEM((rows, 1), jnp.int32),
            pltpu.VMEM((rows, 1), jnp.float32),
            pltpu.VMEM((rows, 1), jnp.float32),
            pltpu.VMEM((rows, 1), jnp.float32),
            pltpu.VMEM((rows, 1), jnp.float32),
        ],
        compiler_params=pltpu.CompilerParams(
            dimension_semantics=("arbitrary",),
        ),
    )(kd, t2, lt)
    return tok.reshape(rows), lp.reshape(rows)
